# baseline (speedup 1.0000x reference)
import functools

import jax
import jax.numpy as jnp
from jax.experimental import pallas as pl
from jax.experimental.pallas import tpu as pltpu

_CHUNK = 1024


def _copy_kernel(n_rows, d, emb_ref, out_ref, bufs, in_sems, out_sems):
    n_chunks = n_rows // _CHUNK

    def in_copy(i):
        return pltpu.make_async_copy(
            emb_ref.at[pl.ds(i * _CHUNK, _CHUNK), :], bufs.at[i], in_sems.at[i]
        )

    def out_copy(i):
        return pltpu.make_async_copy(
            bufs.at[i], out_ref.at[pl.ds(i * _CHUNK, _CHUNK), :], out_sems.at[i]
        )

    for i in range(n_chunks):
        in_copy(i).start()
    for i in range(n_chunks):
        in_copy(i).wait()
        out_copy(i).start()
    for i in range(n_chunks):
        out_copy(i).wait()


def kernel(x, emb):
    n = x.shape[1]
    d = emb.shape[1]
    n_chunks = n // _CHUNK
    return pl.pallas_call(
        functools.partial(_copy_kernel, n, d),
        out_shape=jax.ShapeDtypeStruct((n, d), emb.dtype),
        in_specs=[pl.BlockSpec(memory_space=pl.ANY)],
        out_specs=pl.BlockSpec(memory_space=pl.ANY),
        scratch_shapes=[
            pltpu.VMEM((n_chunks, _CHUNK, d), emb.dtype),
            pltpu.SemaphoreType.DMA((n_chunks,)),
            pltpu.SemaphoreType.DMA((n_chunks,)),
        ],
    )(emb)
